# Initial kernel scaffold; baseline (speedup 1.0000x reference)
import jax
import jax.numpy as jnp
from jax.experimental import pallas as pl


def kernel(x, edge_index, batch, W1a, b1a, W1b, b1b, gamma1, beta1, W2a, b2a, W2b, b2b, gamma2, beta2):
    raise NotImplementedError("write your pallas kernel here")


# speedup vs baseline: 7.5667x; 7.5667x over previous
import functools

import jax
import jax.numpy as jnp
from jax import lax
from jax.experimental import pallas as pl
from jax.experimental.pallas import tpu as pltpu
from jax.experimental.pallas import tpu_sc as plsc

NC = 2
NS = 16
NW = NC * NS
CH = 128
NUM_GRAPHS = 64


@functools.lru_cache(maxsize=None)
def _make_segsum(n_rows_acc, n_feat, k_blocks, ch):
    rt = n_rows_acc // NS
    assert n_rows_acc % NS == 0 and rt % ch == 0
    mesh = plsc.VectorSubcoreMesh(
        core_axis_name="c", subcore_axis_name="s",
        num_cores=NC, num_subcores=NS)

    @functools.partial(
        pl.kernel,
        out_type=jax.ShapeDtypeStruct((NC, n_rows_acc, n_feat), jnp.float32),
        mesh=mesh,
        scratch_types=[
            pltpu.VMEM((k_blocks, 2 * ch), jnp.int32),
            pltpu.VMEM((2 * k_blocks, ch), jnp.int32),
            pltpu.VMEM((ch, n_feat), jnp.float32),
            pltpu.VMEM((ch, n_feat), jnp.float32),
            pltpu.VMEM_SHARED((n_rows_acc, n_feat), jnp.float32),
            pltpu.SemaphoreType.DMA,
            pltpu.SemaphoreType.DMA,
        ],
    )
    def segsum(feat_hbm, src_hbm, dst_hbm, zeros_hbm, out_hbm,
               src_v, dst_v, bufa, bufb, acc, sema, semb):
        cid = lax.axis_index("c")
        sid = lax.axis_index("s")
        wid = sid * NC + cid

        pltpu.sync_copy(src_hbm.at[wid], src_v)
        pltpu.sync_copy(dst_hbm.at[wid], dst_v)

        pltpu.sync_copy(zeros_hbm.at[pl.ds(sid * rt, rt)],
                        acc.at[pl.ds(sid * rt, rt)])
        plsc.subcore_barrier()

        pltpu.async_copy(feat_hbm.at[src_v.at[0, pl.ds(0, ch)]], bufa, sema)

        def step(j, carry):
            jn = jnp.minimum(j + 1, k_blocks - 1)
            ia = src_v.at[j, pl.ds(0, ch)]
            ib = src_v.at[j, pl.ds(ch, ch)]
            inx = src_v.at[jn, pl.ds(0, ch)]
            pltpu.make_async_copy(feat_hbm.at[ia], bufa, sema).wait()
            pltpu.async_copy(feat_hbm.at[ib], bufb, semb)
            pltpu.sync_copy(bufa, acc.at[dst_v.at[2 * j]], add=True)
            pltpu.make_async_copy(feat_hbm.at[ib], bufb, semb).wait()
            pltpu.async_copy(feat_hbm.at[inx], bufa, sema)
            pltpu.sync_copy(bufb, acc.at[dst_v.at[2 * j + 1]], add=True)
            return carry
        lax.fori_loop(0, k_blocks, step, 0)
        pltpu.make_async_copy(feat_hbm.at[src_v.at[0, pl.ds(0, ch)]],
                              bufa, sema).wait()

        plsc.subcore_barrier()
        pltpu.sync_copy(acc.at[pl.ds(sid * rt, rt)],
                        out_hbm.at[cid, pl.ds(sid * rt, rt)])

    return segsum


def _dense_block(xa, Wa, ba, Wb, bb, gamma, beta):
    h = jnp.dot(xa, Wa, preferred_element_type=jnp.float32) + ba
    h = jnp.maximum(h, 0.0)
    h = jnp.dot(h, Wb, preferred_element_type=jnp.float32) + bb
    mu = jnp.mean(h, axis=0, keepdims=True)
    var = jnp.mean((h - mu) * (h - mu), axis=0, keepdims=True)
    h = (h - mu) / jnp.sqrt(var + 1e-5) * gamma + beta
    return jnp.maximum(h, 0.0)


@functools.lru_cache(maxsize=None)
def _make_tc1(n, d, h_dim, n_rows_acc):
    def body(x_ref, agg_ref, Wa_ref, ba_ref, Wb_ref, bb_ref,
             g_ref, be_ref, out_ref):
        xa = x_ref[...] + agg_ref[0, :n, :] + agg_ref[1, :n, :]
        h = _dense_block(xa, Wa_ref[...], ba_ref[...], Wb_ref[...],
                         bb_ref[...], g_ref[...], be_ref[...])
        out_ref[...] = jnp.concatenate(
            [h, jnp.zeros((n, d - h_dim), jnp.float32)], axis=1)

    return pl.pallas_call(
        body, out_shape=jax.ShapeDtypeStruct((n, d), jnp.float32))


@functools.lru_cache(maxsize=None)
def _make_tc2(n, h_dim, n_rows_acc, n_graphs):
    def body(h_ref, agg_ref, batch_ref, Wa_ref, ba_ref, Wb_ref, bb_ref,
             g_ref, be_ref, out_ref):
        xa = (h_ref[:, :h_dim] + agg_ref[0, :n, :h_dim]
              + agg_ref[1, :n, :h_dim])
        h2 = _dense_block(xa, Wa_ref[...], ba_ref[...], Wb_ref[...],
                          bb_ref[...], g_ref[...], be_ref[...])
        gids = lax.broadcasted_iota(jnp.int32, (n_graphs, n), 0)
        mask = (gids == batch_ref[...]).astype(jnp.float32)
        sums = jnp.dot(mask, h2, preferred_element_type=jnp.float32)
        counts = jnp.sum(mask, axis=1, keepdims=True)
        out_ref[...] = sums / jnp.maximum(counts, 1.0)

    return pl.pallas_call(
        body, out_shape=jax.ShapeDtypeStruct((n_graphs, h_dim), jnp.float32))


def kernel(x, edge_index, batch, W1a, b1a, W1b, b1b, gamma1, beta1,
           W2a, b2a, W2b, b2b, gamma2, beta2):
    n, d = x.shape
    h_dim = W1a.shape[1]
    e = edge_index.shape[1]

    ec = 64
    k_blocks = -(-e // (NW * 2 * ec))
    ep = NW * k_blocks * 2 * ec
    n_rows_acc = -(-n // (NS * CH)) * (NS * CH)
    pad = ep - e
    pad_src = lax.iota(jnp.int32, pad) % n
    pad_dst = n + lax.iota(jnp.int32, pad) % (n_rows_acc - n)
    src3 = jnp.concatenate([edge_index[0], pad_src]).reshape(
        NW, k_blocks, 2 * ec)
    dst3 = jnp.concatenate([edge_index[1], pad_dst]).reshape(
        NW, 2 * k_blocks, ec)

    segsum = _make_segsum(n_rows_acc, d, k_blocks, ec)
    tc1 = _make_tc1(n, d, h_dim, n_rows_acc)
    tc2 = _make_tc2(n, h_dim, n_rows_acc, NUM_GRAPHS)

    zeros_acc = jnp.zeros((n_rows_acc, d), jnp.float32)
    agg1p = segsum(x, src3, dst3, zeros_acc)
    hp = tc1(x, agg1p, W1a, b1a.reshape(1, -1), W1b, b1b.reshape(1, -1),
             gamma1.reshape(1, -1), beta1.reshape(1, -1))
    agg2p = segsum(hp, src3, dst3, zeros_acc)
    out = tc2(hp, agg2p, batch.reshape(1, -1), W2a, b2a.reshape(1, -1),
              W2b, b2b.reshape(1, -1), gamma2.reshape(1, -1),
              beta2.reshape(1, -1))
    return out


# speedup vs baseline: 8.1392x; 1.0757x over previous
import functools

import jax
import jax.numpy as jnp
from jax import lax
from jax.experimental import pallas as pl
from jax.experimental.pallas import tpu as pltpu
from jax.experimental.pallas import tpu_sc as plsc

NC = 2
NS = 16
NW = NC * NS
CH = 128
NUM_GRAPHS = 64


@functools.lru_cache(maxsize=None)
def _make_segsum(n_rows_acc, n_feat, k_blocks, ch, tc_tiling=True):
    rt = n_rows_acc // NS
    assert n_rows_acc % NS == 0 and rt % ch == 0
    mesh = plsc.VectorSubcoreMesh(
        core_axis_name="c", subcore_axis_name="s",
        num_cores=NC, num_subcores=NS)

    @functools.partial(
        pl.kernel,
        out_type=jax.ShapeDtypeStruct((NC, n_rows_acc, n_feat), jnp.float32),
        mesh=mesh,
        compiler_params=pltpu.CompilerParams(use_tc_tiling_on_sc=tc_tiling),
        scratch_types=[
            pltpu.VMEM((k_blocks, 2 * ch), jnp.int32),
            pltpu.VMEM((2 * k_blocks, ch), jnp.int32),
            pltpu.VMEM((ch, n_feat), jnp.float32),
            pltpu.VMEM((ch, n_feat), jnp.float32),
            pltpu.VMEM_SHARED((n_rows_acc, n_feat), jnp.float32),
            pltpu.SemaphoreType.DMA,
            pltpu.SemaphoreType.DMA,
        ],
    )
    def segsum(feat_hbm, src_hbm, dst_hbm, zeros_hbm, out_hbm,
               src_v, dst_v, bufa, bufb, acc, sema, semb):
        cid = lax.axis_index("c")
        sid = lax.axis_index("s")
        wid = sid * NC + cid

        pltpu.sync_copy(src_hbm.at[wid], src_v)
        pltpu.sync_copy(dst_hbm.at[wid], dst_v)

        pltpu.sync_copy(zeros_hbm.at[pl.ds(sid * rt, rt)],
                        acc.at[pl.ds(sid * rt, rt)])
        plsc.subcore_barrier()

        pltpu.async_copy(feat_hbm.at[src_v.at[0, pl.ds(0, ch)]], bufa, sema)

        def step(j, carry):
            jn = jnp.minimum(j + 1, k_blocks - 1)
            ia = src_v.at[j, pl.ds(0, ch)]
            ib = src_v.at[j, pl.ds(ch, ch)]
            inx = src_v.at[jn, pl.ds(0, ch)]
            pltpu.make_async_copy(feat_hbm.at[ia], bufa, sema).wait()
            pltpu.async_copy(feat_hbm.at[ib], bufb, semb)
            pltpu.sync_copy(bufa, acc.at[dst_v.at[2 * j]], add=True)
            pltpu.make_async_copy(feat_hbm.at[ib], bufb, semb).wait()
            pltpu.async_copy(feat_hbm.at[inx], bufa, sema)
            pltpu.sync_copy(bufb, acc.at[dst_v.at[2 * j + 1]], add=True)
            return carry
        lax.fori_loop(0, k_blocks, step, 0)
        pltpu.make_async_copy(feat_hbm.at[src_v.at[0, pl.ds(0, ch)]],
                              bufa, sema).wait()

        plsc.subcore_barrier()
        pltpu.sync_copy(acc.at[pl.ds(sid * rt, rt)],
                        out_hbm.at[cid, pl.ds(sid * rt, rt)])

    return segsum


def _dense_block(xa, Wa, ba, Wb, bb, gamma, beta):
    h = jnp.dot(xa, Wa, preferred_element_type=jnp.float32) + ba
    h = jnp.maximum(h, 0.0)
    h = jnp.dot(h, Wb, preferred_element_type=jnp.float32) + bb
    mu = jnp.mean(h, axis=0, keepdims=True)
    var = jnp.mean((h - mu) * (h - mu), axis=0, keepdims=True)
    h = (h - mu) / jnp.sqrt(var + 1e-5) * gamma + beta
    return jnp.maximum(h, 0.0)


@functools.lru_cache(maxsize=None)
def _make_tc1(n, d, h_dim, n_rows_acc):
    def body(x_ref, agg_ref, Wa_ref, ba_ref, Wb_ref, bb_ref,
             g_ref, be_ref, out_ref):
        xa = x_ref[...] + agg_ref[0, :n, :] + agg_ref[1, :n, :]
        out_ref[...] = _dense_block(xa, Wa_ref[...], ba_ref[...], Wb_ref[...],
                                    bb_ref[...], g_ref[...], be_ref[...])

    return pl.pallas_call(
        body, out_shape=jax.ShapeDtypeStruct((n, h_dim), jnp.float32))


@functools.lru_cache(maxsize=None)
def _make_tc2(n, h_dim, n_rows_acc, n_graphs):
    def body(h_ref, agg_ref, batch_ref, Wa_ref, ba_ref, Wb_ref, bb_ref,
             g_ref, be_ref, out_ref):
        xa = h_ref[...] + agg_ref[0, :n, :] + agg_ref[1, :n, :]
        h2 = _dense_block(xa, Wa_ref[...], ba_ref[...], Wb_ref[...],
                          bb_ref[...], g_ref[...], be_ref[...])
        gids = lax.broadcasted_iota(jnp.int32, (n_graphs, n), 0)
        mask = (gids == batch_ref[...]).astype(jnp.float32)
        sums = jnp.dot(mask, h2, preferred_element_type=jnp.float32)
        counts = jnp.sum(mask, axis=1, keepdims=True)
        out_ref[...] = sums / jnp.maximum(counts, 1.0)

    return pl.pallas_call(
        body, out_shape=jax.ShapeDtypeStruct((n_graphs, h_dim), jnp.float32))


def kernel(x, edge_index, batch, W1a, b1a, W1b, b1b, gamma1, beta1,
           W2a, b2a, W2b, b2b, gamma2, beta2):
    n, d = x.shape
    h_dim = W1a.shape[1]
    e = edge_index.shape[1]

    ec = 64
    k_blocks = -(-e // (NW * 2 * ec))
    ep = NW * k_blocks * 2 * ec
    n_rows_acc = -(-n // (NS * CH)) * (NS * CH)
    pad = ep - e
    pad_src = lax.iota(jnp.int32, pad) % n
    pad_dst = n + lax.iota(jnp.int32, pad) % (n_rows_acc - n)
    src3 = jnp.concatenate([edge_index[0], pad_src]).reshape(
        NW, k_blocks, 2 * ec)
    dst3 = jnp.concatenate([edge_index[1], pad_dst]).reshape(
        NW, 2 * k_blocks, ec)

    segsum1 = _make_segsum(n_rows_acc, d, k_blocks, ec, True)
    segsum2 = _make_segsum(n_rows_acc, h_dim, k_blocks, ec, False)
    tc1 = _make_tc1(n, d, h_dim, n_rows_acc)
    tc2 = _make_tc2(n, h_dim, n_rows_acc, NUM_GRAPHS)

    agg1p = segsum1(x, src3, dst3, jnp.zeros((n_rows_acc, d), jnp.float32))
    hp = tc1(x, agg1p, W1a, b1a.reshape(1, -1), W1b, b1b.reshape(1, -1),
             gamma1.reshape(1, -1), beta1.reshape(1, -1))
    agg2p = segsum2(hp, src3, dst3,
                    jnp.zeros((n_rows_acc, h_dim), jnp.float32))
    out = tc2(hp, agg2p, batch.reshape(1, -1), W2a, b2a.reshape(1, -1),
              W2b, b2b.reshape(1, -1), gamma2.reshape(1, -1),
              beta2.reshape(1, -1))
    return out


# speedup vs baseline: 10.5481x; 1.2960x over previous
import functools

import jax
import jax.numpy as jnp
from jax import lax
from jax.experimental import pallas as pl
from jax.experimental.pallas import tpu as pltpu
from jax.experimental.pallas import tpu_sc as plsc

NC = 2
NS = 16
NW = NC * NS
CH = 128
NUM_GRAPHS = 64


@functools.lru_cache(maxsize=None)
def _make_segsum(n_rows_acc, n_feat, m_chunks, ch, n_stage, tc_tiling):
    rt = n_rows_acc // NS
    mg = m_chunks // n_stage
    assert n_rows_acc % NS == 0 and m_chunks % n_stage == 0 and mg % 2 == 0
    mesh = plsc.VectorSubcoreMesh(
        core_axis_name="c", subcore_axis_name="s",
        num_cores=NC, num_subcores=NS)

    @functools.partial(
        pl.kernel,
        out_type=jax.ShapeDtypeStruct((NC, n_rows_acc, n_feat), jnp.float32),
        mesh=mesh,
        compiler_params=pltpu.CompilerParams(use_tc_tiling_on_sc=tc_tiling),
        scratch_types=[
            pltpu.VMEM((mg, ch), jnp.int32),
            pltpu.VMEM((mg, ch), jnp.int32),
            pltpu.VMEM((ch, n_feat), jnp.float32),
            pltpu.VMEM((ch, n_feat), jnp.float32),
            pltpu.VMEM_SHARED((n_rows_acc, n_feat), jnp.float32),
            pltpu.SemaphoreType.DMA,
            pltpu.SemaphoreType.DMA,
        ],
    )
    def segsum(feat_hbm, src_hbm, dst_hbm, zeros_hbm, out_hbm,
               src_v, dst_v, bufa, bufb, acc, sema, semb):
        cid = lax.axis_index("c")
        sid = lax.axis_index("s")
        wid = sid * NC + cid

        pltpu.sync_copy(zeros_hbm.at[pl.ds(sid * rt, rt)],
                        acc.at[pl.ds(sid * rt, rt)])
        plsc.subcore_barrier()

        for g in range(n_stage):
            pltpu.sync_copy(src_hbm.at[wid, pl.ds(g * mg, mg)], src_v)
            pltpu.sync_copy(dst_hbm.at[wid, pl.ds(g * mg, mg)], dst_v)

            pltpu.async_copy(feat_hbm.at[src_v.at[0]], bufa, sema)

            def step(j2, carry):
                ja = 2 * j2
                jb = ja + 1
                jn = jnp.minimum(ja + 2, mg - 1)
                pltpu.make_async_copy(
                    feat_hbm.at[src_v.at[ja]], bufa, sema).wait()
                pltpu.async_copy(feat_hbm.at[src_v.at[jb]], bufb, semb)
                pltpu.sync_copy(bufa, acc.at[dst_v.at[ja]], add=True)
                pltpu.make_async_copy(
                    feat_hbm.at[src_v.at[jb]], bufb, semb).wait()
                pltpu.async_copy(feat_hbm.at[src_v.at[jn]], bufa, sema)
                pltpu.sync_copy(bufb, acc.at[dst_v.at[jb]], add=True)
                return carry
            lax.fori_loop(0, mg // 2, step, 0)
            pltpu.make_async_copy(feat_hbm.at[src_v.at[0]], bufa, sema).wait()

        plsc.subcore_barrier()
        pltpu.sync_copy(acc.at[pl.ds(sid * rt, rt)],
                        out_hbm.at[cid, pl.ds(sid * rt, rt)])

    return segsum


def _dense_block(xa, Wa, ba, Wb, bb, gamma, beta):
    h = jnp.dot(xa, Wa, preferred_element_type=jnp.float32) + ba
    h = jnp.maximum(h, 0.0)
    h = jnp.dot(h, Wb, preferred_element_type=jnp.float32) + bb
    mu = jnp.mean(h, axis=0, keepdims=True)
    var = jnp.mean((h - mu) * (h - mu), axis=0, keepdims=True)
    h = (h - mu) / jnp.sqrt(var + 1e-5) * gamma + beta
    return jnp.maximum(h, 0.0)


@functools.lru_cache(maxsize=None)
def _make_tc1(n, d, h_dim, n_rows_acc):
    def body(x_ref, agg_ref, Wa_ref, ba_ref, Wb_ref, bb_ref,
             g_ref, be_ref, out_ref):
        xa = x_ref[...] + agg_ref[0, :n, :] + agg_ref[1, :n, :]
        out_ref[...] = _dense_block(xa, Wa_ref[...], ba_ref[...], Wb_ref[...],
                                    bb_ref[...], g_ref[...], be_ref[...])

    return pl.pallas_call(
        body, out_shape=jax.ShapeDtypeStruct((n, h_dim), jnp.float32))


@functools.lru_cache(maxsize=None)
def _make_tc2(n, h_dim, n_rows_acc, n_graphs):
    def body(h_ref, agg_ref, batch_ref, Wa_ref, ba_ref, Wb_ref, bb_ref,
             g_ref, be_ref, out_ref):
        xa = h_ref[...] + agg_ref[0, :n, :] + agg_ref[1, :n, :]
        h2 = _dense_block(xa, Wa_ref[...], ba_ref[...], Wb_ref[...],
                          bb_ref[...], g_ref[...], be_ref[...])
        gids = lax.broadcasted_iota(jnp.int32, (n_graphs, n), 0)
        mask = (gids == batch_ref[...]).astype(jnp.float32)
        sums = jnp.dot(mask, h2, preferred_element_type=jnp.float32)
        counts = jnp.sum(mask, axis=1, keepdims=True)
        out_ref[...] = sums / jnp.maximum(counts, 1.0)

    return pl.pallas_call(
        body, out_shape=jax.ShapeDtypeStruct((n_graphs, h_dim), jnp.float32))


def kernel(x, edge_index, batch, W1a, b1a, W1b, b1b, gamma1, beta1,
           W2a, b2a, W2b, b2b, gamma2, beta2):
    n, d = x.shape
    h_dim = W1a.shape[1]
    e = edge_index.shape[1]

    ec = 128
    m_chunks = -(-e // (NW * ec))
    if m_chunks % 4:
        m_chunks += 4 - m_chunks % 4
    ep = NW * m_chunks * ec
    n_rows_acc = -(-n // (NS * CH)) * (NS * CH)
    pad = ep - e
    pad_src = lax.iota(jnp.int32, pad) % n
    pad_dst = n + lax.iota(jnp.int32, pad) % (n_rows_acc - n)
    src3 = jnp.concatenate([edge_index[0], pad_src]).reshape(NW, m_chunks, ec)
    dst3 = jnp.concatenate([edge_index[1], pad_dst]).reshape(NW, m_chunks, ec)

    segsum1 = _make_segsum(n_rows_acc, d, m_chunks, ec, 2, True)
    segsum2 = _make_segsum(n_rows_acc, h_dim, m_chunks, ec, 1, False)
    tc1 = _make_tc1(n, d, h_dim, n_rows_acc)
    tc2 = _make_tc2(n, h_dim, n_rows_acc, NUM_GRAPHS)

    agg1p = segsum1(x, src3, dst3, jnp.zeros((n_rows_acc, d), jnp.float32))
    hp = tc1(x, agg1p, W1a, b1a.reshape(1, -1), W1b, b1b.reshape(1, -1),
             gamma1.reshape(1, -1), beta1.reshape(1, -1))
    agg2p = segsum2(hp, src3, dst3,
                    jnp.zeros((n_rows_acc, h_dim), jnp.float32))
    out = tc2(hp, agg2p, batch.reshape(1, -1), W2a, b2a.reshape(1, -1),
              W2b, b2b.reshape(1, -1), gamma2.reshape(1, -1),
              beta2.reshape(1, -1))
    return out
